# baseline (speedup 1.0000x reference)
import jax
import jax.numpy as jnp
from jax.experimental import pallas as pl
from jax.experimental.pallas import tpu as pltpu

_N = 32768
_ROWS = 256
_COLS = 128
_NCAND = 1024
_R = 8
_K = 96
_P = 0.9
_LN_P = -0.10536051565782628


def _prefilter(x):
    rowio = jax.lax.broadcasted_iota(jnp.int32, (_ROWS, _COLS), 0)
    lane = jax.lax.broadcasted_iota(jnp.int32, (1, _COLS), 1)
    neg_inf = jnp.float32(-jnp.inf)
    cms, idxs = [], []
    for _ in range(_R):
        cm = jnp.max(x, axis=0, keepdims=True)
        rr = jnp.min(
            jnp.where(x == cm, rowio, jnp.int32(_ROWS)), axis=0, keepdims=True
        )
        cms.append(cm)
        idxs.append((rr * _COLS + lane).astype(jnp.float32))
        x = jnp.where(rowio == rr, neg_inf, x)
    vrow = jnp.concatenate(cms, axis=1)
    irow = jnp.concatenate(idxs, axis=1)
    vstack = jnp.concatenate(cms, axis=0)
    istack = jnp.concatenate(idxs, axis=0)
    both = jnp.concatenate([vstack, istack], axis=0)
    ident = jnp.where(
        jax.lax.broadcasted_iota(jnp.int32, (_COLS, _COLS), 0)
        == jax.lax.broadcasted_iota(jnp.int32, (_COLS, _COLS), 1),
        jnp.float32(1.0),
        jnp.float32(0.0),
    )
    tr = jax.lax.dot_general(
        ident, both, (((1,), (1,)), ((), ())),
        preferred_element_type=jnp.float32,
    )
    vcol = jnp.concatenate([tr[:, k:k + 1] for k in range(_R)], axis=0)
    icol = jnp.concatenate([tr[:, _R + k:_R + k + 1] for k in range(_R)], axis=0)
    return vrow, irow, vcol, icol


def _gt(v_a, i_a, v_b, i_b):
    return (v_a > v_b) | ((v_a == v_b) & (i_a < i_b))


def _rank95_key(r, v, i):
    sel = r == jnp.int32(_K - 1)
    tv = jnp.sum(jnp.where(sel, v, jnp.float32(0.0)), keepdims=True).reshape(1, 1)
    ti = jnp.sum(jnp.where(sel, i, jnp.float32(0.0)), keepdims=True).reshape(1, 1)
    return tv, ti


def _full_rank_is95(x, flatf, tv, ti):
    cnt = jnp.sum(
        jnp.where(_gt(x, flatf, tv, ti), jnp.int32(1), jnp.int32(0))
    )
    return cnt == jnp.int32(_K - 1)


def _slow_loss(p_in, t_in, p_buf, t_buf):
    p_buf[...] = p_in[...]
    t_buf[...] = t_in[...]
    flat_idx = (
        jax.lax.broadcasted_iota(jnp.int32, (_ROWS, _COLS), 0) * _COLS
        + jax.lax.broadcasted_iota(jnp.int32, (_ROWS, _COLS), 1)
    )
    neg_inf = jnp.float32(-jnp.inf)
    big = jnp.int32(_N)

    def extract(buf_ref):
        x = buf_ref[...]
        m = jnp.max(x, axis=(0, 1), keepdims=True)
        i = jnp.min(jnp.where(x == m, flat_idx, big), axis=(0, 1), keepdims=True)
        buf_ref[...] = jnp.where(flat_idx == i, neg_inf, x)
        return i

    def body(k, carry):
        acc, w = carry
        t_idx = extract(t_buf)
        p_idx = extract(p_buf)
        acc = acc + jnp.where(t_idx == p_idx, w, jnp.float32(0.0))
        return acc, w * jnp.float32(_P)

    acc, _ = jax.lax.fori_loop(
        0, _K, body,
        (jnp.zeros((1, 1), jnp.float32), jnp.full((1, 1), 1.0 - _P, jnp.float32)),
    )
    return jnp.float32(1.0) - acc[0, 0]


def _rbo_kernel(p_in, t_in, out_ref, p_buf, t_buf):
    t = t_in[...]
    p = p_in[...]
    flatf = (
        jax.lax.broadcasted_iota(jnp.int32, (_ROWS, _COLS), 0) * _COLS
        + jax.lax.broadcasted_iota(jnp.int32, (_ROWS, _COLS), 1)
    ).astype(jnp.float32)

    vrow_t, irow_t, vcol_t, icol_t = _prefilter(t)
    vrow_p, irow_p, vcol_p, icol_p = _prefilter(p)

    one = jnp.int32(1)
    zero = jnp.int32(0)
    g_t = _gt(vrow_t, irow_t, vcol_t, icol_t)
    r_t = jnp.sum(jnp.where(g_t, one, zero), axis=1, keepdims=True)
    h_p = _gt(vcol_p, icol_p, vrow_p, irow_p)
    r_p = jnp.sum(jnp.where(h_p, one, zero), axis=0, keepdims=True)

    tv_t, ti_t = _rank95_key(r_t, vcol_t, icol_t)
    tv_p, ti_p = _rank95_key(r_p, vrow_p, irow_p)
    ok_t = _full_rank_is95(t, flatf, tv_t, ti_t)
    ok_p = _full_rank_is95(p, flatf, tv_p, ti_p)
    ok = jnp.logical_and(ok_t, ok_p)

    match = (r_t == r_p) & (icol_t == irow_p) & (r_t < jnp.int32(_K))
    w = jnp.float32(1.0 - _P) * jnp.exp(
        jnp.float32(_LN_P) * r_t.astype(jnp.float32)
    )
    rbo = jnp.sum(jnp.where(match, w, jnp.float32(0.0)))
    loss_fast = jnp.float32(1.0) - rbo

    @pl.when(ok)
    def _():
        out_ref[0, 0] = loss_fast

    @pl.when(jnp.logical_not(ok))
    def _():
        out_ref[0, 0] = _slow_loss(p_in, t_in, p_buf, t_buf)


@jax.jit
def kernel(predictions, targets):
    p2 = predictions.reshape(_ROWS, _COLS)
    t2 = targets.reshape(_ROWS, _COLS)
    out = pl.pallas_call(
        _rbo_kernel,
        out_shape=jax.ShapeDtypeStruct((1, 1), jnp.float32),
        in_specs=[
            pl.BlockSpec(memory_space=pltpu.VMEM),
            pl.BlockSpec(memory_space=pltpu.VMEM),
        ],
        out_specs=pl.BlockSpec(memory_space=pltpu.SMEM),
        scratch_shapes=[
            pltpu.VMEM((_ROWS, _COLS), jnp.float32),
            pltpu.VMEM((_ROWS, _COLS), jnp.float32),
        ],
    )(p2, t2)
    return out[0, 0]


# speedup vs baseline: 9.8333x; 9.8333x over previous
import jax
import jax.numpy as jnp
from jax.experimental import pallas as pl
from jax.experimental.pallas import tpu as pltpu

_N = 32768
_ROWS = 256
_COLS = 128
_NCAND = 1024
_R = 8
_K = 96
_P = 0.9
_LN_P = -0.10536051565782628


def _prefilter(x):
    rowio = jax.lax.broadcasted_iota(jnp.int32, (_ROWS, _COLS), 0)
    lane = jax.lax.broadcasted_iota(jnp.int32, (1, _COLS), 1)
    neg_inf = jnp.float32(-jnp.inf)
    cms, idxs = [], []
    for _ in range(_R):
        cm = jnp.max(x, axis=0, keepdims=True)
        rr = jnp.min(
            jnp.where(x == cm, rowio, jnp.int32(_ROWS)), axis=0, keepdims=True
        )
        cms.append(cm)
        idxs.append((rr * _COLS + lane).astype(jnp.float32))
        x = jnp.where(rowio == rr, neg_inf, x)
    vrow = jnp.concatenate(cms, axis=1)
    irow = jnp.concatenate(idxs, axis=1)
    eq = (
        jax.lax.broadcasted_iota(jnp.int32, (_NCAND, 1), 0)
        == jax.lax.broadcasted_iota(jnp.int32, (1, _NCAND), 1)
    )
    vcol = jnp.sum(
        jnp.where(eq, vrow, jnp.float32(0.0)), axis=1, keepdims=True
    )
    icol = jnp.sum(
        jnp.where(eq, irow, jnp.float32(0.0)), axis=1, keepdims=True
    )
    return vrow, irow, vcol, icol


def _gt(v_a, i_a, v_b, i_b):
    return (v_a > v_b) | ((v_a == v_b) & (i_a < i_b))


def _rank95_key(r, v, i):
    sel = r == jnp.int32(_K - 1)
    tv = jnp.sum(jnp.where(sel, v, jnp.float32(0.0)), keepdims=True).reshape(1, 1)
    ti = jnp.sum(jnp.where(sel, i, jnp.float32(0.0)), keepdims=True).reshape(1, 1)
    return tv, ti


def _full_rank_is95(x, flatf, tv, ti):
    cnt = jnp.sum(
        jnp.where(_gt(x, flatf, tv, ti), jnp.int32(1), jnp.int32(0))
    )
    return cnt == jnp.int32(_K - 1)


def _slow_loss(p_in, t_in, p_buf, t_buf):
    p_buf[...] = p_in[...]
    t_buf[...] = t_in[...]
    flat_idx = (
        jax.lax.broadcasted_iota(jnp.int32, (_ROWS, _COLS), 0) * _COLS
        + jax.lax.broadcasted_iota(jnp.int32, (_ROWS, _COLS), 1)
    )
    neg_inf = jnp.float32(-jnp.inf)
    big = jnp.int32(_N)

    def extract(buf_ref):
        x = buf_ref[...]
        m = jnp.max(x, axis=(0, 1), keepdims=True)
        i = jnp.min(jnp.where(x == m, flat_idx, big), axis=(0, 1), keepdims=True)
        buf_ref[...] = jnp.where(flat_idx == i, neg_inf, x)
        return i

    def body(k, carry):
        acc, w = carry
        t_idx = extract(t_buf)
        p_idx = extract(p_buf)
        acc = acc + jnp.where(t_idx == p_idx, w, jnp.float32(0.0))
        return acc, w * jnp.float32(_P)

    acc, _ = jax.lax.fori_loop(
        0, _K, body,
        (jnp.zeros((1, 1), jnp.float32), jnp.full((1, 1), 1.0 - _P, jnp.float32)),
    )
    return jnp.float32(1.0) - acc[0, 0]


def _rbo_kernel(p_in, t_in, out_ref, p_buf, t_buf):
    t = t_in[...]
    p = p_in[...]
    flatf = (
        jax.lax.broadcasted_iota(jnp.int32, (_ROWS, _COLS), 0) * _COLS
        + jax.lax.broadcasted_iota(jnp.int32, (_ROWS, _COLS), 1)
    ).astype(jnp.float32)

    vrow_t, irow_t, vcol_t, icol_t = _prefilter(t)
    vrow_p, irow_p, vcol_p, icol_p = _prefilter(p)

    one = jnp.int32(1)
    zero = jnp.int32(0)
    g_t = _gt(vrow_t, irow_t, vcol_t, icol_t)
    r_t = jnp.sum(jnp.where(g_t, one, zero), axis=1, keepdims=True)
    h_p = _gt(vcol_p, icol_p, vrow_p, irow_p)
    r_p = jnp.sum(jnp.where(h_p, one, zero), axis=0, keepdims=True)

    tv_t, ti_t = _rank95_key(r_t, vcol_t, icol_t)
    tv_p, ti_p = _rank95_key(r_p, vrow_p, irow_p)
    ok_t = _full_rank_is95(t, flatf, tv_t, ti_t)
    ok_p = _full_rank_is95(p, flatf, tv_p, ti_p)
    ok = jnp.logical_and(ok_t, ok_p)

    match = (r_t == r_p) & (icol_t == irow_p) & (r_t < jnp.int32(_K))
    w = jnp.float32(1.0 - _P) * jnp.exp(
        jnp.float32(_LN_P) * r_t.astype(jnp.float32)
    )
    rbo = jnp.sum(jnp.where(match, w, jnp.float32(0.0)))
    loss_fast = jnp.float32(1.0) - rbo

    @pl.when(ok)
    def _():
        out_ref[0, 0] = loss_fast

    @pl.when(jnp.logical_not(ok))
    def _():
        out_ref[0, 0] = _slow_loss(p_in, t_in, p_buf, t_buf)


@jax.jit
def kernel(predictions, targets):
    p2 = predictions.reshape(_ROWS, _COLS)
    t2 = targets.reshape(_ROWS, _COLS)
    out = pl.pallas_call(
        _rbo_kernel,
        out_shape=jax.ShapeDtypeStruct((1, 1), jnp.float32),
        in_specs=[
            pl.BlockSpec(memory_space=pltpu.VMEM),
            pl.BlockSpec(memory_space=pltpu.VMEM),
        ],
        out_specs=pl.BlockSpec(memory_space=pltpu.SMEM),
        scratch_shapes=[
            pltpu.VMEM((_ROWS, _COLS), jnp.float32),
            pltpu.VMEM((_ROWS, _COLS), jnp.float32),
        ],
    )(p2, t2)
    return out[0, 0]
